# baseline (speedup 1.0000x reference)
import functools

import jax
import jax.numpy as jnp
import numpy as np
from jax import lax
from jax.experimental import pallas as pl
from jax.experimental.pallas import tpu as pltpu
from jax.experimental.pallas import tpu_sc as plsc

_INFO = plsc.get_sparse_core_info()
_NC, _NS, _L = _INFO.num_cores, _INFO.num_subcores, _INFO.num_lanes
_NW = _NC * _NS


def _pos_encoding(seq_len, d):
    pos = np.arange(seq_len, dtype=np.float32)[:, None]
    div_term = np.exp(np.arange(0, d, 2, dtype=np.float32) * (-np.log(10000.0) / d))
    enc = np.zeros((seq_len, d), dtype=np.float32)
    enc[:, 0::2] = np.sin(pos * div_term)
    enc[:, 1::2] = np.cos(pos * div_term)
    return jnp.asarray(enc)


@functools.partial(jax.jit, static_argnames=("seq", "chunk"))
def _emb_lookup(table, idx_flat, pos, *, seq, chunk):
    n = idx_flat.shape[0]
    d = table.shape[1]
    per_w = n // _NW
    n_chunks = per_w // chunk
    scale = float(np.sqrt(d))
    mesh = plsc.VectorSubcoreMesh(core_axis_name="c", subcore_axis_name="s")

    @functools.partial(
        pl.kernel,
        mesh=mesh,
        compiler_params=pltpu.CompilerParams(use_tc_tiling_on_sc=False),
        out_type=jax.ShapeDtypeStruct((n, d), jnp.float32),
        scratch_types=[
            pltpu.VMEM((chunk,), jnp.int32),
            pltpu.VMEM((chunk, d), jnp.float32),
            pltpu.VMEM((seq, d), jnp.float32),
            pltpu.SemaphoreType.DMA,
        ],
    )
    def body(table_hbm, idx_hbm, pos_hbm, out_hbm, idx_v, rows_v, pos_v, sem):
        wid = lax.axis_index("s") * _NC + lax.axis_index("c")
        base = wid * per_w
        pltpu.sync_copy(pos_hbm, pos_v)

        def do_chunk(c, _):
            start = base + c * chunk
            pltpu.sync_copy(idx_hbm.at[pl.ds(start, chunk)], idx_v)
            pltpu.async_copy(table_hbm.at[idx_v], rows_v, sem).wait()

            def do_row(r, _):
                p = lax.rem(r, seq)
                for j in range(d // _L):
                    sl = pl.ds(j * _L, _L)
                    rows_v[r, sl] = rows_v[r, sl] * scale + pos_v[p, sl]
                return 0

            lax.fori_loop(0, chunk, do_row, 0)
            pltpu.sync_copy(rows_v, out_hbm.at[pl.ds(start, chunk)])
            return 0

        lax.fori_loop(0, n_chunks, do_chunk, 0)

    return body(table, idx_flat, pos)


def kernel(x, table):
    b, s = x.shape
    d = table.shape[1]
    pos = _pos_encoding(s, d)
    out = _emb_lookup(table, x.reshape(-1), pos, seq=s, chunk=2 * s)
    return out.reshape(b, s, d)


# speedup vs baseline: 1.1570x; 1.1570x over previous
import functools

import jax
import jax.numpy as jnp
import numpy as np
from jax import lax
from jax.experimental import pallas as pl
from jax.experimental.pallas import tpu as pltpu
from jax.experimental.pallas import tpu_sc as plsc

_INFO = plsc.get_sparse_core_info()
_NC, _NS, _L = _INFO.num_cores, _INFO.num_subcores, _INFO.num_lanes
_NW = _NC * _NS


def _pos_encoding(seq_len, d):
    pos = np.arange(seq_len, dtype=np.float32)[:, None]
    div_term = np.exp(np.arange(0, d, 2, dtype=np.float32) * (-np.log(10000.0) / d))
    enc = np.zeros((seq_len, d), dtype=np.float32)
    enc[:, 0::2] = np.sin(pos * div_term)
    enc[:, 1::2] = np.cos(pos * div_term)
    return jnp.asarray(enc.reshape(-1))


@functools.partial(jax.jit, static_argnames=("seq", "d"))
def _emb_lookup(table, x, pos, *, seq, d):
    b = x.shape[0]
    seq_w = b // _NW
    scale = float(np.sqrt(d))
    nj = d // _L
    s0n = (seq // 2 + 7) & ~7
    s1n = seq - s0n
    mesh = plsc.VectorSubcoreMesh(core_axis_name="c", subcore_axis_name="s")

    @functools.partial(
        pl.kernel,
        mesh=mesh,
        compiler_params=pltpu.CompilerParams(use_tc_tiling_on_sc=False),
        out_type=jax.ShapeDtypeStruct((b, seq, d), jnp.float32),
        scratch_types=[
            pltpu.VMEM((seq_w, seq), jnp.int32),
            pltpu.VMEM((seq, d), jnp.float32),
            pltpu.VMEM((seq, d), jnp.float32),
            pltpu.VMEM((seq, d), jnp.float32),
            pltpu.VMEM((seq, d), jnp.float32),
            pltpu.VMEM((seq * d,), jnp.float32),
            pltpu.SemaphoreType.DMA,
            pltpu.SemaphoreType.DMA,
            pltpu.SemaphoreType.DMA,
            pltpu.SemaphoreType.DMA,
        ],
    )
    def body(table_hbm, x_hbm, pos_hbm, out_hbm,
             idx_v, in0, in1, ou0, ou1, pos_v, g0, g1, st0, st1):
        wid = lax.axis_index("s") * _NC + lax.axis_index("c")
        base = wid * seq_w
        pltpu.sync_copy(pos_hbm, pos_v)
        pltpu.sync_copy(x_hbm.at[pl.ds(base, seq_w), :], idx_v)

        def start_gather(c, ibuf, sem):
            pltpu.async_copy(
                table_hbm.at[idx_v.at[c, pl.ds(0, s0n)]],
                ibuf.at[pl.ds(0, s0n)], sem)
            pltpu.async_copy(
                table_hbm.at[idx_v.at[c, pl.ds(s0n, s1n)]],
                ibuf.at[pl.ds(s0n, s1n)], sem)

        def wait_gather(ibuf, sem):
            pltpu.make_async_copy(out_hbm.at[0], ibuf, sem).wait()

        def start_store(c, obuf, sem):
            pltpu.async_copy(obuf, out_hbm.at[base + c], sem)

        def wait_store(obuf, sem):
            pltpu.make_async_copy(obuf, out_hbm.at[0], sem).wait()

        def compute(ibuf, obuf):
            def row(r, _):
                pbase = r * d
                for j in range(nj):
                    sl = pl.ds(_L * j, _L)
                    obuf[r, sl] = (ibuf[r, sl] * scale
                                   + pos_v[pl.ds(pbase + _L * j, _L)])
                return 0

            lax.fori_loop(0, seq, row, 0)

        start_gather(0, in0, g0)

        def pair(p, _):
            sub = (
                (0, in0, ou0, g0, st0, in1, g1),
                (1, in1, ou1, g1, st1, in0, g0),
            )
            for off, ibuf, obuf, gs, ss, nxt_in, nxt_gs in sub:
                c = 2 * p + off
                wait_gather(ibuf, gs)
                if off == 0:
                    start_gather(c + 1, nxt_in, nxt_gs)
                else:
                    @pl.when(p < seq_w // 2 - 1)
                    def _():
                        start_gather(c + 1, nxt_in, nxt_gs)

                @pl.when(p >= 1)
                def _():
                    wait_store(obuf, ss)

                compute(ibuf, obuf)
                start_store(c, obuf, ss)
            return 0

        lax.fori_loop(0, seq_w // 2, pair, 0)
        wait_store(ou0, st0)
        wait_store(ou1, st1)

    return body(table, x, pos)


def kernel(x, table):
    b, s = x.shape
    d = table.shape[1]
    pos = _pos_encoding(s, d)
    return _emb_lookup(table, x, pos, seq=s, d=d)
